# baseline (speedup 1.0000x reference)
import jax
import jax.numpy as jnp
from jax import lax
from jax.experimental import pallas as pl
from jax.experimental.pallas import tpu as pltpu
from jax.experimental.pallas import tpu_sc as plsc

N = 10000
NP = 10240
E = 320000
NT = 16
EPT = E // NT
NCH = EPT // 16
NSL = NP // NT
NG = 64
EPS = 0.1


def _tanh2(z2):
    return 1.0 - 2.0 / (jnp.exp(z2) + 1.0)


def _rsqrt(d):
    i = plsc.bitcast(d, jnp.int32)
    i = jnp.int32(0x5F3759DF) - (i >> 1)
    y = plsc.bitcast(i, jnp.float32)
    for _ in range(3):
        y = y * (1.5 - 0.5 * d * y * y)
    return y


def _body(xp, ei0, ei1, batchp, wv, bv, al1, ar1, al2, ar2, al3, ar3,
          out, hpartp, hpartq, hbm_p, hbm_q,
          rows, cols, pacc, qacc, stage, dinvf, pfull, qfull,
          xs, bs, initp, initq, wb, red16, pv, qv, cv, obuf, poolall, sem,
          ):
    wid = lax.axis_index("s")
    ebase = wid * EPT
    nb = wid * NSL

    zero16 = jnp.zeros((16,), jnp.float32)
    one16 = jnp.full((16,), 1.0, jnp.float32)

    pltpu.sync_copy(ei0.at[pl.ds(ebase, EPT)], rows)
    pltpu.sync_copy(ei1.at[pl.ds(ebase, EPT)], cols)
    for i, src in enumerate([wv, bv, al1, ar1, al2, ar2, al3, ar3]):
        pltpu.sync_copy(src, wb.at[i])
    pltpu.sync_copy(xp.at[pl.ds(nb, NSL)], xs)
    pltpu.sync_copy(batchp.at[pl.ds(nb, NSL)], bs)
    pltpu.sync_copy(xp, pfull)

    def fill(ref, n, v16):
        def bd(i, c):
            for u in range(8):
                ref[pl.ds(128 * i + 16 * u, 16)] = v16
            return c
        lax.fori_loop(0, n // 128, bd, 0)

    fill(qfull, NP, one16)

    def dot2(i, j):
        acc = jnp.zeros((16,), jnp.float32)
        for c in range(8):
            acc = acc + wb[i, pl.ds(16 * c, 16)] * wb[j, pl.ds(16 * c, 16)]
        lanes = lax.iota(jnp.int32, 16)
        for sh in (8, 4, 2, 1):
            red16[pl.ds(0, 16)] = acc
            acc = acc + plsc.load_gather(red16, [lanes ^ sh])
        return acc + acc

    coefs = []
    for k in range(3):
        coefs.append((dot2(0, 2 + 2 * k), dot2(1, 2 + 2 * k),
                      dot2(0, 3 + 2 * k), dot2(1, 3 + 2 * k)))

    def write_partial(acc_ref, hpart):
        pltpu.sync_copy(acc_ref, hpart.at[pl.ds(wid * NP, NP)])

    def read_stage(hpart):
        for c in range(NT):
            pltpu.async_copy(hpart.at[pl.ds(c * NP + nb, NSL)], stage.at[c],
                             sem)
        for c in range(NT):
            pltpu.make_async_copy(hpart.at[pl.ds(c * NP + nb, NSL)],
                                  stage.at[c], sem).wait()

    def reduce_stage(ch):
        s = stage[0, pl.ds(16 * ch, 16)]
        for c in range(1, NT):
            s = s + stage[c, pl.ds(16 * ch, 16)]
        return s

    def add_reduced(dst):
        def bd(ch, c):
            sl = pl.ds(16 * ch, 16)
            dst[sl] = dst[sl] + reduce_stage(ch)
            return c
        lax.fori_loop(0, NSL // 16, bd, 0)

    fill(pacc, NP, zero16)

    @plsc.parallel_loop(0, NCH, unroll=8)
    def _(i):
        ci = cols[pl.ds(16 * i, 16)]
        plsc.addupdate_scatter(pacc, [ci], one16)
    write_partial(pacc, hpartp)
    plsc.subcore_barrier()
    read_stage(hpartp)

    def dinv_chunk(ch, c):
        deg = reduce_stage(ch) + 1.0
        initq[pl.ds(16 * ch, 16)] = _rsqrt(deg)
        return c
    lax.fori_loop(0, NSL // 16, dinv_chunk, 0)
    pltpu.sync_copy(initq, hbm_p.at[pl.ds(nb, NSL)])
    plsc.subcore_barrier()
    pltpu.sync_copy(hbm_p, dinvf)

    for k in range(3):
        wl2, bl2, wr2, br2 = coefs[k]

        fill(pacc, NP, zero16)
        fill(qacc, NP, zero16)

        @plsc.parallel_loop(0, NCH, unroll=4)
        def _(i):
            sl = pl.ds(16 * i, 16)
            r = rows[sl]
            ci = cols[sl]
            pj = plsc.load_gather(pfull, [r])
            qj = plsc.load_gather(qfull, [r])
            pi = plsc.load_gather(pfull, [ci])
            qi = plsc.load_gather(qfull, [ci])
            dr = plsc.load_gather(dinvf, [r])
            dc = plsc.load_gather(dinvf, [ci])
            z2 = (pj * wl2 + qj * bl2) + (pi * wr2 + qi * br2)
            a = _tanh2(z2) * (dr * dc)
            plsc.addupdate_scatter(pacc, [ci], a * pj)
            plsc.addupdate_scatter(qacc, [ci], a * qj)
        wp = pltpu.async_copy(pacc, hpartp.at[pl.ds(wid * NP, NP)], sem)
        wq = pltpu.async_copy(qacc, hpartq.at[pl.ds(wid * NP, NP)], sem)

        def init_chunk(ch, c):
            sl = pl.ds(16 * ch, 16)
            pld = pfull[pl.ds(nb + 16 * ch, 16)]
            qld = qfull[pl.ds(nb + 16 * ch, 16)]
            dv = dinvf[pl.ds(nb + 16 * ch, 16)]
            z2 = (pld * wl2 + qld * bl2) + (pld * wr2 + qld * br2)
            a = _tanh2(z2) * dv * dv
            initp[sl] = a * pld + EPS * xs[sl]
            initq[sl] = a * qld + EPS
            return c
        lax.fori_loop(0, NSL // 16, init_chunk, 0)
        wp.wait()
        wq.wait()
        plsc.subcore_barrier()

        read_stage(hpartp)
        add_reduced(initp)
        read_stage(hpartq)
        add_reduced(initq)

        if k < 2:
            s1 = pltpu.async_copy(initp, hbm_p.at[pl.ds(nb, NSL)], sem)
            s2 = pltpu.async_copy(initq, hbm_q.at[pl.ds(nb, NSL)], sem)
            s1.wait()
            s2.wait()
            plsc.subcore_barrier()
            r1 = pltpu.async_copy(hbm_p, pfull, sem)
            r2 = pltpu.async_copy(hbm_q, qfull, sem)
            r1.wait()
            r2.wait()

    for c in range(80 // 16):
        pv[pl.ds(16 * c, 16)] = zero16
        qv[pl.ds(16 * c, 16)] = zero16
        cv[pl.ds(16 * c, 16)] = zero16

    def pool_chunk(i, c):
        sl = pl.ds(16 * i, 16)
        b16 = bs[sl]
        plsc.addupdate_scatter(pv, [b16], initp[sl])
        plsc.addupdate_scatter(qv, [b16], initq[sl])
        plsc.addupdate_scatter(cv, [b16], one16)
        return c
    lax.fori_loop(0, NSL // 16, pool_chunk, 0)

    for c in range(5):
        poolall[pl.ds(16 * c, 16)] = pv[pl.ds(16 * c, 16)]
        poolall[pl.ds(80 + 16 * c, 16)] = qv[pl.ds(16 * c, 16)]
        poolall[pl.ds(160 + 16 * c, 16)] = cv[pl.ds(16 * c, 16)]
    pltpu.sync_copy(poolall.at[pl.ds(0, 240)],
                    hpartp.at[pl.ds(wid * 240, 240)])
    plsc.subcore_barrier()
    pltpu.sync_copy(hpartp.at[pl.ds(0, NT * 240)], poolall)

    for c in range(5):
        sl = pl.ds(16 * c, 16)
        sp = zero16
        sq = zero16
        sc_ = zero16
        for t in range(NT):
            sp = sp + poolall[pl.ds(t * 240 + 16 * c, 16)]
            sq = sq + poolall[pl.ds(t * 240 + 80 + 16 * c, 16)]
            sc_ = sc_ + poolall[pl.ds(t * 240 + 160 + 16 * c, 16)]
        pv[sl] = sp
        qv[sl] = sq
        cv[sl] = sc_

    for g4 in range(4):
        g = jnp.full((16,), 1, jnp.int32) * (wid * 4 + g4)
        Pb = plsc.load_gather(pv, [g])
        Qb = plsc.load_gather(qv, [g])
        Cb = jnp.maximum(plsc.load_gather(cv, [g]), 1.0)
        Pn = Pb / Cb
        Qn = Qb / Cb
        for c in range(8):
            sl = pl.ds(16 * c, 16)
            obuf[g4, sl] = Pn * wb[0, sl] + Qn * wb[1, sl]
    pltpu.sync_copy(obuf, out.at[pl.ds(wid * 4, 4)])


@jax.jit
def _run(xp, ei0, ei1, batchp, wv, bv, al1, ar1, al2, ar2, al3, ar3):
    mesh = plsc.VectorSubcoreMesh(core_axis_name="c", subcore_axis_name="s",
                                  num_cores=1)
    f = pl.kernel(
        _body,
        out_type=[
            jax.ShapeDtypeStruct((NG, 128), jnp.float32),
            jax.ShapeDtypeStruct((NT * NP,), jnp.float32),
            jax.ShapeDtypeStruct((NT * NP,), jnp.float32),
            jax.ShapeDtypeStruct((NP,), jnp.float32),
            jax.ShapeDtypeStruct((NP,), jnp.float32),
        ],
        mesh=mesh,
        compiler_params=pltpu.CompilerParams(needs_layout_passes=False),
        scratch_types=[
            pltpu.VMEM((EPT,), jnp.int32),
            pltpu.VMEM((EPT,), jnp.int32),
            pltpu.VMEM((NP,), jnp.float32),
            pltpu.VMEM((NP,), jnp.float32),
            pltpu.VMEM((NT, NSL), jnp.float32),
            pltpu.VMEM((NP,), jnp.float32),
            pltpu.VMEM((NP,), jnp.float32),
            pltpu.VMEM((NP,), jnp.float32),
            pltpu.VMEM((NSL,), jnp.float32),
            pltpu.VMEM((NSL,), jnp.int32),
            pltpu.VMEM((NSL,), jnp.float32),
            pltpu.VMEM((NSL,), jnp.float32),
            pltpu.VMEM((8, 128), jnp.float32),
            pltpu.VMEM((128,), jnp.float32),
            pltpu.VMEM((80,), jnp.float32),
            pltpu.VMEM((80,), jnp.float32),
            pltpu.VMEM((80,), jnp.float32),
            pltpu.VMEM((4, 128), jnp.float32),
            pltpu.VMEM((NT * 240,), jnp.float32),
            pltpu.SemaphoreType.DMA,
        ],
    )
    outs = f(xp, ei0, ei1, batchp, wv, bv, al1, ar1, al2, ar2, al3, ar3)
    return outs[0]


def kernel(x, edge_index, batch, lin_w, lin_b,
           att_l1, att_r1, att_l2, att_r2, att_l3, att_r3):
    xp = jnp.pad(x[:, 0], (0, NP - N))
    ei = edge_index.astype(jnp.int32)
    batchp = jnp.pad(batch.astype(jnp.int32), (0, NP - N),
                     constant_values=NG)
    return _run(xp, ei[0], ei[1], batchp, lin_w[:, 0], lin_b,
                att_l1, att_r1, att_l2, att_r2, att_l3, att_r3)


# speedup vs baseline: 2.0308x; 2.0308x over previous
import jax
import jax.numpy as jnp
from jax import lax
from jax.experimental import pallas as pl
from jax.experimental.pallas import tpu as pltpu
from jax.experimental.pallas import tpu_sc as plsc

N = 10000
NP = 10240
E = 320000
NT = 16
EPT = E // NT
NCH = EPT // 16
NSL = NP // NT
NG = 64
EPS = 0.1


def _tanh2(z2):
    return 1.0 - 2.0 / (jnp.exp(z2) + 1.0)


def _rsqrt(d):
    i = plsc.bitcast(d, jnp.int32)
    i = jnp.int32(0x5F3759DF) - (i >> 1)
    y = plsc.bitcast(i, jnp.float32)
    for _ in range(3):
        y = y * (1.5 - 0.5 * d * y * y)
    return y


def _body(xp, ei0, ei1, batchp, wv, bv, al1, ar1, al2, ar2, al3, ar3,
          out, hpartp, hpartq, hbm_p, hbm_q,
          rows, cols, pacc, qacc, stage, dinvf, pfull, qfull,
          xs, bs, initp, initq, wb, red16, pv, qv, cv, obuf, poolall, sem,
          ):
    wid = lax.axis_index("s")
    ebase = wid * EPT
    nb = wid * NSL

    zero16 = jnp.zeros((16,), jnp.float32)
    one16 = jnp.full((16,), 1.0, jnp.float32)

    pltpu.sync_copy(ei0.at[pl.ds(ebase, EPT)], rows)
    pltpu.sync_copy(ei1.at[pl.ds(ebase, EPT)], cols)
    for i, src in enumerate([wv, bv, al1, ar1, al2, ar2, al3, ar3]):
        pltpu.sync_copy(src, wb.at[i])
    pltpu.sync_copy(xp.at[pl.ds(nb, NSL)], xs)
    pltpu.sync_copy(batchp.at[pl.ds(nb, NSL)], bs)
    pltpu.sync_copy(xp, pfull)

    def fill(ref, n, v16):
        def bd(i, c):
            for u in range(8):
                ref[pl.ds(128 * i + 16 * u, 16)] = v16
            return c
        lax.fori_loop(0, n // 128, bd, 0)

    fill(qfull, NP, one16)

    def dot2(i, j):
        acc = jnp.zeros((16,), jnp.float32)
        for c in range(8):
            acc = acc + wb[i, pl.ds(16 * c, 16)] * wb[j, pl.ds(16 * c, 16)]
        lanes = lax.iota(jnp.int32, 16)
        for sh in (8, 4, 2, 1):
            red16[pl.ds(0, 16)] = acc
            acc = acc + plsc.load_gather(red16, [lanes ^ sh])
        return acc + acc

    coefs = []
    for k in range(3):
        coefs.append((dot2(0, 2 + 2 * k), dot2(1, 2 + 2 * k),
                      dot2(0, 3 + 2 * k), dot2(1, 3 + 2 * k)))

    def write_partial(acc_ref, hpart):
        pltpu.sync_copy(acc_ref, hpart.at[pl.ds(wid * NP, NP)])

    def read_stage(hpart):
        for c in range(NT):
            pltpu.async_copy(hpart.at[pl.ds(c * NP + nb, NSL)], stage.at[c],
                             sem)
        for c in range(NT):
            pltpu.make_async_copy(hpart.at[pl.ds(c * NP + nb, NSL)],
                                  stage.at[c], sem).wait()

    def reduce_stage(ch):
        s = stage[0, pl.ds(16 * ch, 16)]
        for c in range(1, NT):
            s = s + stage[c, pl.ds(16 * ch, 16)]
        return s

    def add_reduced(dst):
        @plsc.parallel_loop(0, NSL // 16, unroll=2)
        def _(ch):
            sl = pl.ds(16 * ch, 16)
            dst[sl] = dst[sl] + reduce_stage(ch)

    fill(pacc, NP, zero16)

    @plsc.parallel_loop(0, NCH, unroll=8)
    def _(i):
        ci = cols[pl.ds(16 * i, 16)]
        plsc.addupdate_scatter(pacc, [ci], one16)
    write_partial(pacc, hpartp)
    plsc.subcore_barrier()
    read_stage(hpartp)

    @plsc.parallel_loop(0, NSL // 16, unroll=2)
    def _(ch):
        deg = reduce_stage(ch) + 1.0
        initq[pl.ds(16 * ch, 16)] = _rsqrt(deg)
    pltpu.sync_copy(initq, hbm_p.at[pl.ds(nb, NSL)])
    plsc.subcore_barrier()
    pltpu.sync_copy(hbm_p, dinvf)

    for k in range(3):
        wl2, bl2, wr2, br2 = coefs[k]

        fill(pacc, NP, zero16)
        fill(qacc, NP, zero16)

        @plsc.parallel_loop(0, NCH, unroll=8)
        def _(i):
            sl = pl.ds(16 * i, 16)
            r = rows[sl]
            ci = cols[sl]
            pj = plsc.load_gather(pfull, [r])
            qj = plsc.load_gather(qfull, [r])
            pi = plsc.load_gather(pfull, [ci])
            qi = plsc.load_gather(qfull, [ci])
            dr = plsc.load_gather(dinvf, [r])
            dc = plsc.load_gather(dinvf, [ci])
            z2 = (pj * wl2 + qj * bl2) + (pi * wr2 + qi * br2)
            a = _tanh2(z2) * (dr * dc)
            plsc.addupdate_scatter(pacc, [ci], a * pj)
            plsc.addupdate_scatter(qacc, [ci], a * qj)
        wp = pltpu.async_copy(pacc, hpartp.at[pl.ds(wid * NP, NP)], sem)
        wq = pltpu.async_copy(qacc, hpartq.at[pl.ds(wid * NP, NP)], sem)

        @plsc.parallel_loop(0, NSL // 16, unroll=4)
        def _(ch):
            sl = pl.ds(16 * ch, 16)
            pld = pfull[pl.ds(nb + 16 * ch, 16)]
            qld = qfull[pl.ds(nb + 16 * ch, 16)]
            dv = dinvf[pl.ds(nb + 16 * ch, 16)]
            z2 = (pld * wl2 + qld * bl2) + (pld * wr2 + qld * br2)
            a = _tanh2(z2) * dv * dv
            initp[sl] = a * pld + EPS * xs[sl]
            initq[sl] = a * qld + EPS
        wp.wait()
        wq.wait()
        plsc.subcore_barrier()

        read_stage(hpartp)
        add_reduced(initp)
        read_stage(hpartq)
        add_reduced(initq)

        if k < 2:
            s1 = pltpu.async_copy(initp, hbm_p.at[pl.ds(nb, NSL)], sem)
            s2 = pltpu.async_copy(initq, hbm_q.at[pl.ds(nb, NSL)], sem)
            s1.wait()
            s2.wait()
            plsc.subcore_barrier()
            r1 = pltpu.async_copy(hbm_p, pfull, sem)
            r2 = pltpu.async_copy(hbm_q, qfull, sem)
            r1.wait()
            r2.wait()

    for c in range(80 // 16):
        pv[pl.ds(16 * c, 16)] = zero16
        qv[pl.ds(16 * c, 16)] = zero16
        cv[pl.ds(16 * c, 16)] = zero16

    @plsc.parallel_loop(0, NSL // 16, unroll=4)
    def _(i):
        sl = pl.ds(16 * i, 16)
        b16 = bs[sl]
        plsc.addupdate_scatter(pv, [b16], initp[sl])
        plsc.addupdate_scatter(qv, [b16], initq[sl])
        plsc.addupdate_scatter(cv, [b16], one16)

    for c in range(5):
        poolall[pl.ds(16 * c, 16)] = pv[pl.ds(16 * c, 16)]
        poolall[pl.ds(80 + 16 * c, 16)] = qv[pl.ds(16 * c, 16)]
        poolall[pl.ds(160 + 16 * c, 16)] = cv[pl.ds(16 * c, 16)]
    pltpu.sync_copy(poolall.at[pl.ds(0, 240)],
                    hpartp.at[pl.ds(wid * 240, 240)])
    plsc.subcore_barrier()
    pltpu.sync_copy(hpartp.at[pl.ds(0, NT * 240)], poolall)

    for c in range(5):
        sl = pl.ds(16 * c, 16)
        sp = zero16
        sq = zero16
        sc_ = zero16
        for t in range(NT):
            sp = sp + poolall[pl.ds(t * 240 + 16 * c, 16)]
            sq = sq + poolall[pl.ds(t * 240 + 80 + 16 * c, 16)]
            sc_ = sc_ + poolall[pl.ds(t * 240 + 160 + 16 * c, 16)]
        pv[sl] = sp
        qv[sl] = sq
        cv[sl] = sc_

    for g4 in range(4):
        g = jnp.full((16,), 1, jnp.int32) * (wid * 4 + g4)
        Pb = plsc.load_gather(pv, [g])
        Qb = plsc.load_gather(qv, [g])
        Cb = jnp.maximum(plsc.load_gather(cv, [g]), 1.0)
        Pn = Pb / Cb
        Qn = Qb / Cb
        for c in range(8):
            sl = pl.ds(16 * c, 16)
            obuf[g4, sl] = Pn * wb[0, sl] + Qn * wb[1, sl]
    pltpu.sync_copy(obuf, out.at[pl.ds(wid * 4, 4)])


@jax.jit
def _run(xp, ei0, ei1, batchp, wv, bv, al1, ar1, al2, ar2, al3, ar3):
    mesh = plsc.VectorSubcoreMesh(core_axis_name="c", subcore_axis_name="s",
                                  num_cores=1)
    f = pl.kernel(
        _body,
        out_type=[
            jax.ShapeDtypeStruct((NG, 128), jnp.float32),
            jax.ShapeDtypeStruct((NT * NP,), jnp.float32),
            jax.ShapeDtypeStruct((NT * NP,), jnp.float32),
            jax.ShapeDtypeStruct((NP,), jnp.float32),
            jax.ShapeDtypeStruct((NP,), jnp.float32),
        ],
        mesh=mesh,
        compiler_params=pltpu.CompilerParams(needs_layout_passes=False),
        scratch_types=[
            pltpu.VMEM((EPT,), jnp.int32),
            pltpu.VMEM((EPT,), jnp.int32),
            pltpu.VMEM((NP,), jnp.float32),
            pltpu.VMEM((NP,), jnp.float32),
            pltpu.VMEM((NT, NSL), jnp.float32),
            pltpu.VMEM((NP,), jnp.float32),
            pltpu.VMEM((NP,), jnp.float32),
            pltpu.VMEM((NP,), jnp.float32),
            pltpu.VMEM((NSL,), jnp.float32),
            pltpu.VMEM((NSL,), jnp.int32),
            pltpu.VMEM((NSL,), jnp.float32),
            pltpu.VMEM((NSL,), jnp.float32),
            pltpu.VMEM((8, 128), jnp.float32),
            pltpu.VMEM((128,), jnp.float32),
            pltpu.VMEM((80,), jnp.float32),
            pltpu.VMEM((80,), jnp.float32),
            pltpu.VMEM((80,), jnp.float32),
            pltpu.VMEM((4, 128), jnp.float32),
            pltpu.VMEM((NT * 240,), jnp.float32),
            pltpu.SemaphoreType.DMA,
        ],
    )
    outs = f(xp, ei0, ei1, batchp, wv, bv, al1, ar1, al2, ar2, al3, ar3)
    return outs[0]


def kernel(x, edge_index, batch, lin_w, lin_b,
           att_l1, att_r1, att_l2, att_r2, att_l3, att_r3):
    xp = jnp.pad(x[:, 0], (0, NP - N))
    ei = edge_index.astype(jnp.int32)
    batchp = jnp.pad(batch.astype(jnp.int32), (0, NP - N),
                     constant_values=NG)
    return _run(xp, ei[0], ei[1], batchp, lin_w[:, 0], lin_b,
                att_l1, att_r1, att_l2, att_r2, att_l3, att_r3)


# speedup vs baseline: 2.1940x; 1.0804x over previous
import jax
import jax.numpy as jnp
from jax import lax
from jax.experimental import pallas as pl
from jax.experimental.pallas import tpu as pltpu
from jax.experimental.pallas import tpu_sc as plsc

N = 10000
NP = 10240
E = 320000
NT = 16
EPT = E // NT
NCH = EPT // 16
NSL = NP // NT
NG = 64
EPS = 0.1


def _tanh2(z2):
    return 1.0 - 2.0 / (jnp.exp(z2) + 1.0)


def _rsqrt(d):
    i = plsc.bitcast(d, jnp.int32)
    i = jnp.int32(0x5F3759DF) - (i >> 1)
    y = plsc.bitcast(i, jnp.float32)
    for _ in range(3):
        y = y * (1.5 - 0.5 * d * y * y)
    return y


def _body(xp, ei0, ei1, batchp, wv, bv, al1, ar1, al2, ar2, al3, ar3,
          out, hpartp, hpartq, hbm_p, hbm_q,
          rows, cols, pacc, qacc, stage, dinvf, pfull, qfull,
          xs, bs, initp, initq, wb, red16, pv, qv, cv, obuf, poolall, sem,
          ):
    wid = lax.axis_index("s")
    ebase = wid * EPT
    nb = wid * NSL

    zero16 = jnp.zeros((16,), jnp.float32)
    one16 = jnp.full((16,), 1.0, jnp.float32)

    setup_copies = [
        (ei0.at[pl.ds(ebase, EPT)], rows),
        (ei1.at[pl.ds(ebase, EPT)], cols),
        (xp.at[pl.ds(nb, NSL)], xs),
        (batchp.at[pl.ds(nb, NSL)], bs),
        (xp, pfull),
    ]
    for i, src in enumerate([wv, bv, al1, ar1, al2, ar2, al3, ar3]):
        setup_copies.append((src, wb.at[i]))
    handles = [pltpu.async_copy(a, b, sem) for a, b in setup_copies]
    for h in handles:
        h.wait()

    def fill(ref, n, v16):
        def bd(i, c):
            for u in range(8):
                ref[pl.ds(128 * i + 16 * u, 16)] = v16
            return c
        lax.fori_loop(0, n // 128, bd, 0)

    fill(qfull, NP, one16)

    def dot2(i, j):
        acc = jnp.zeros((16,), jnp.float32)
        for c in range(8):
            acc = acc + wb[i, pl.ds(16 * c, 16)] * wb[j, pl.ds(16 * c, 16)]
        lanes = lax.iota(jnp.int32, 16)
        for sh in (8, 4, 2, 1):
            red16[pl.ds(0, 16)] = acc
            acc = acc + plsc.load_gather(red16, [lanes ^ sh])
        return acc + acc

    coefs = []
    for k in range(3):
        coefs.append((dot2(0, 2 + 2 * k), dot2(1, 2 + 2 * k),
                      dot2(0, 3 + 2 * k), dot2(1, 3 + 2 * k)))

    def write_partial(acc_ref, hpart):
        pltpu.sync_copy(acc_ref, hpart.at[pl.ds(wid * NP, NP)])

    def read_stage(hpart, base=0):
        for c in range(NT):
            pltpu.async_copy(hpart.at[pl.ds(c * NP + nb, NSL)],
                             stage.at[base + c], sem)

    def drain_stage(hpart, base=0):
        for c in range(NT):
            pltpu.make_async_copy(hpart.at[pl.ds(c * NP + nb, NSL)],
                                  stage.at[base + c], sem).wait()

    def reduce_stage(ch, base=0):
        s = stage[base, pl.ds(16 * ch, 16)]
        for c in range(1, NT):
            s = s + stage[base + c, pl.ds(16 * ch, 16)]
        return s

    def add_reduced(dst, base=0):
        @plsc.parallel_loop(0, NSL // 16, unroll=2)
        def _(ch):
            sl = pl.ds(16 * ch, 16)
            dst[sl] = dst[sl] + reduce_stage(ch, base)

    fill(pacc, NP, zero16)

    @plsc.parallel_loop(0, NCH, unroll=8)
    def _(i):
        ci = cols[pl.ds(16 * i, 16)]
        plsc.addupdate_scatter(pacc, [ci], one16)
    write_partial(pacc, hpartp)
    plsc.subcore_barrier()
    read_stage(hpartp)
    drain_stage(hpartp)

    @plsc.parallel_loop(0, NSL // 16, unroll=2)
    def _(ch):
        deg = reduce_stage(ch) + 1.0
        initq[pl.ds(16 * ch, 16)] = _rsqrt(deg)
    pltpu.sync_copy(initq, hbm_p.at[pl.ds(nb, NSL)])
    plsc.subcore_barrier()
    pltpu.sync_copy(hbm_p, dinvf)

    for k in range(3):
        wl2, bl2, wr2, br2 = coefs[k]

        fill(pacc, NP, zero16)
        fill(qacc, NP, zero16)

        @plsc.parallel_loop(0, NCH, unroll=8)
        def _(i):
            sl = pl.ds(16 * i, 16)
            r = rows[sl]
            ci = cols[sl]
            pj = plsc.load_gather(pfull, [r])
            qj = plsc.load_gather(qfull, [r])
            pi = plsc.load_gather(pfull, [ci])
            qi = plsc.load_gather(qfull, [ci])
            dr = plsc.load_gather(dinvf, [r])
            dc = plsc.load_gather(dinvf, [ci])
            z2 = (pj * wl2 + qj * bl2) + (pi * wr2 + qi * br2)
            a = _tanh2(z2) * (dr * dc)
            plsc.addupdate_scatter(pacc, [ci], a * pj)
            plsc.addupdate_scatter(qacc, [ci], a * qj)
        wp = pltpu.async_copy(pacc, hpartp.at[pl.ds(wid * NP, NP)], sem)
        wq = pltpu.async_copy(qacc, hpartq.at[pl.ds(wid * NP, NP)], sem)

        @plsc.parallel_loop(0, NSL // 16, unroll=4)
        def _(ch):
            sl = pl.ds(16 * ch, 16)
            pld = pfull[pl.ds(nb + 16 * ch, 16)]
            qld = qfull[pl.ds(nb + 16 * ch, 16)]
            dv = dinvf[pl.ds(nb + 16 * ch, 16)]
            z2 = (pld * wl2 + qld * bl2) + (pld * wr2 + qld * br2)
            a = _tanh2(z2) * dv * dv
            initp[sl] = a * pld + EPS * xs[sl]
            initq[sl] = a * qld + EPS
        wp.wait()
        wq.wait()
        plsc.subcore_barrier()

        read_stage(hpartp, 0)
        read_stage(hpartq, NT)
        drain_stage(hpartp, 0)
        drain_stage(hpartq, NT)
        add_reduced(initp, 0)
        add_reduced(initq, NT)

        if k < 2:
            s1 = pltpu.async_copy(initp, hbm_p.at[pl.ds(nb, NSL)], sem)
            s2 = pltpu.async_copy(initq, hbm_q.at[pl.ds(nb, NSL)], sem)
            s1.wait()
            s2.wait()
            plsc.subcore_barrier()
            r1 = pltpu.async_copy(hbm_p, pfull, sem)
            r2 = pltpu.async_copy(hbm_q, qfull, sem)
            r1.wait()
            r2.wait()

    for c in range(80 // 16):
        pv[pl.ds(16 * c, 16)] = zero16
        qv[pl.ds(16 * c, 16)] = zero16
        cv[pl.ds(16 * c, 16)] = zero16

    @plsc.parallel_loop(0, NSL // 16, unroll=4)
    def _(i):
        sl = pl.ds(16 * i, 16)
        b16 = bs[sl]
        plsc.addupdate_scatter(pv, [b16], initp[sl])
        plsc.addupdate_scatter(qv, [b16], initq[sl])
        plsc.addupdate_scatter(cv, [b16], one16)

    for c in range(5):
        poolall[pl.ds(16 * c, 16)] = pv[pl.ds(16 * c, 16)]
        poolall[pl.ds(80 + 16 * c, 16)] = qv[pl.ds(16 * c, 16)]
        poolall[pl.ds(160 + 16 * c, 16)] = cv[pl.ds(16 * c, 16)]
    pltpu.sync_copy(poolall.at[pl.ds(0, 240)],
                    hpartp.at[pl.ds(wid * 240, 240)])
    plsc.subcore_barrier()
    pltpu.sync_copy(hpartp.at[pl.ds(0, NT * 240)], poolall)

    for c in range(5):
        sl = pl.ds(16 * c, 16)
        sp = zero16
        sq = zero16
        sc_ = zero16
        for t in range(NT):
            sp = sp + poolall[pl.ds(t * 240 + 16 * c, 16)]
            sq = sq + poolall[pl.ds(t * 240 + 80 + 16 * c, 16)]
            sc_ = sc_ + poolall[pl.ds(t * 240 + 160 + 16 * c, 16)]
        pv[sl] = sp
        qv[sl] = sq
        cv[sl] = sc_

    for g4 in range(4):
        g = jnp.full((16,), 1, jnp.int32) * (wid * 4 + g4)
        Pb = plsc.load_gather(pv, [g])
        Qb = plsc.load_gather(qv, [g])
        Cb = jnp.maximum(plsc.load_gather(cv, [g]), 1.0)
        Pn = Pb / Cb
        Qn = Qb / Cb
        for c in range(8):
            sl = pl.ds(16 * c, 16)
            obuf[g4, sl] = Pn * wb[0, sl] + Qn * wb[1, sl]
    pltpu.sync_copy(obuf, out.at[pl.ds(wid * 4, 4)])


@jax.jit
def _run(xp, ei0, ei1, batchp, wv, bv, al1, ar1, al2, ar2, al3, ar3):
    mesh = plsc.VectorSubcoreMesh(core_axis_name="c", subcore_axis_name="s",
                                  num_cores=1)
    f = pl.kernel(
        _body,
        out_type=[
            jax.ShapeDtypeStruct((NG, 128), jnp.float32),
            jax.ShapeDtypeStruct((NT * NP,), jnp.float32),
            jax.ShapeDtypeStruct((NT * NP,), jnp.float32),
            jax.ShapeDtypeStruct((NP,), jnp.float32),
            jax.ShapeDtypeStruct((NP,), jnp.float32),
        ],
        mesh=mesh,
        compiler_params=pltpu.CompilerParams(needs_layout_passes=False),
        scratch_types=[
            pltpu.VMEM((EPT,), jnp.int32),
            pltpu.VMEM((EPT,), jnp.int32),
            pltpu.VMEM((NP,), jnp.float32),
            pltpu.VMEM((NP,), jnp.float32),
            pltpu.VMEM((2 * NT, NSL), jnp.float32),
            pltpu.VMEM((NP,), jnp.float32),
            pltpu.VMEM((NP,), jnp.float32),
            pltpu.VMEM((NP,), jnp.float32),
            pltpu.VMEM((NSL,), jnp.float32),
            pltpu.VMEM((NSL,), jnp.int32),
            pltpu.VMEM((NSL,), jnp.float32),
            pltpu.VMEM((NSL,), jnp.float32),
            pltpu.VMEM((8, 128), jnp.float32),
            pltpu.VMEM((128,), jnp.float32),
            pltpu.VMEM((80,), jnp.float32),
            pltpu.VMEM((80,), jnp.float32),
            pltpu.VMEM((80,), jnp.float32),
            pltpu.VMEM((4, 128), jnp.float32),
            pltpu.VMEM((NT * 240,), jnp.float32),
            pltpu.SemaphoreType.DMA,
        ],
    )
    outs = f(xp, ei0, ei1, batchp, wv, bv, al1, ar1, al2, ar2, al3, ar3)
    return outs[0]


def kernel(x, edge_index, batch, lin_w, lin_b,
           att_l1, att_r1, att_l2, att_r2, att_l3, att_r3):
    xp = jnp.pad(x[:, 0], (0, NP - N))
    ei = edge_index.astype(jnp.int32)
    batchp = jnp.pad(batch.astype(jnp.int32), (0, NP - N),
                     constant_values=NG)
    return _run(xp, ei[0], ei[1], batchp, lin_w[:, 0], lin_b,
                att_l1, att_r1, att_l2, att_r2, att_l3, att_r3)
